# initial kernel scaffold (unmeasured)
import jax
import jax.numpy as jnp
from jax import lax
from jax.experimental import pallas as pl
from jax.experimental.pallas import tpu as pltpu

N_DEV = 32
M = 8192
N_OUT = 4096
SEG = M // N_DEV


def kernel(x, w_mat):
    m, k_per = x.shape
    _, n = w_mat.shape

    def body(x_ref, w_ref, out_ref,
             acc_ref, rs_recv_ref, ag_send_ref, ag_recv_ref,
             rs_send_sem, rs_recv_sem, ag_send_sem, ag_recv_sem,
             rs_credit, ag_credit, copy_sem):
        my = lax.axis_index("i")
        left = lax.rem(my + N_DEV - 1, N_DEV)
        right = lax.rem(my + 1, N_DEV)

        barrier = pltpu.get_barrier_semaphore()
        pl.semaphore_signal(barrier, inc=1, device_id=(left,),
                            device_id_type=pl.DeviceIdType.MESH)
        pl.semaphore_signal(barrier, inc=1, device_id=(right,),
                            device_id_type=pl.DeviceIdType.MESH)
        pl.semaphore_wait(barrier, 2)

        pl.semaphore_signal(rs_credit, inc=1, device_id=(left,),
                            device_id_type=pl.DeviceIdType.MESH)
        pl.semaphore_signal(ag_credit, inc=1, device_id=(left,),
                            device_id_type=pl.DeviceIdType.MESH)

        def partial(seg):
            return jnp.dot(x_ref[pl.ds(seg * SEG, SEG), :], w_ref[...],
                           preferred_element_type=jnp.float32)

        acc_ref[...] = partial(my)

        def rs_step(t, carry):
            pl.semaphore_wait(rs_credit, 1)
            rdma = pltpu.make_async_remote_copy(
                src_ref=acc_ref, dst_ref=rs_recv_ref,
                send_sem=rs_send_sem, recv_sem=rs_recv_sem,
                device_id=(right,), device_id_type=pl.DeviceIdType.MESH,
            )
            rdma.start()
            seg = lax.rem(my - t - 1 + 2 * N_DEV, N_DEV)
            part = partial(seg)
            rdma.wait()
            acc_ref[...] = rs_recv_ref[...] + part

            @pl.when(t < N_DEV - 2)
            def _():
                pl.semaphore_signal(rs_credit, inc=1, device_id=(left,),
                                    device_id_type=pl.DeviceIdType.MESH)
            return carry

        lax.fori_loop(0, N_DEV - 1, rs_step, 0)

        my_seg = lax.rem(my + 1, N_DEV)
        ag_send_ref[...] = jnp.maximum(acc_ref[...], 0.0)
        copy = pltpu.make_async_copy(
            ag_send_ref, out_ref.at[pl.ds(my_seg * SEG, SEG), :], copy_sem)
        copy.start()
        copy.wait()

        def ag_step(t, carry):
            pl.semaphore_wait(ag_credit, 1)
            rdma = pltpu.make_async_remote_copy(
                src_ref=ag_send_ref, dst_ref=ag_recv_ref,
                send_sem=ag_send_sem, recv_sem=ag_recv_sem,
                device_id=(right,), device_id_type=pl.DeviceIdType.MESH,
            )
            rdma.start()
            rdma.wait()
            s_recv = lax.rem(my - t + 2 * N_DEV, N_DEV)
            ag_send_ref[...] = ag_recv_ref[...]
            cp = pltpu.make_async_copy(
                ag_recv_ref, out_ref.at[pl.ds(s_recv * SEG, SEG), :], copy_sem)
            cp.start()
            cp.wait()

            @pl.when(t < N_DEV - 2)
            def _():
                pl.semaphore_signal(ag_credit, inc=1, device_id=(left,),
                                    device_id_type=pl.DeviceIdType.MESH)
            return carry

        lax.fori_loop(0, N_DEV - 1, ag_step, 0)

    return pl.pallas_call(
        body,
        out_shape=jax.ShapeDtypeStruct((M, N_OUT), jnp.float32),
        in_specs=[
            pl.BlockSpec(memory_space=pltpu.VMEM),
            pl.BlockSpec(memory_space=pltpu.VMEM),
        ],
        out_specs=pl.BlockSpec(memory_space=pltpu.ANY),
        scratch_shapes=[
            pltpu.VMEM((SEG, N_OUT), jnp.float32),
            pltpu.VMEM((SEG, N_OUT), jnp.float32),
            pltpu.VMEM((SEG, N_OUT), jnp.float32),
            pltpu.VMEM((SEG, N_OUT), jnp.float32),
            pltpu.SemaphoreType.DMA,
            pltpu.SemaphoreType.DMA,
            pltpu.SemaphoreType.DMA,
            pltpu.SemaphoreType.DMA,
            pltpu.SemaphoreType.REGULAR,
            pltpu.SemaphoreType.REGULAR,
            pltpu.SemaphoreType.DMA,
        ],
        compiler_params=pltpu.CompilerParams(collective_id=0),
    )(x, w_mat)


# baseline (device time: 3399564 ns/iter reference)
import jax
import jax.numpy as jnp
from jax import lax
from jax.experimental import pallas as pl
from jax.experimental.pallas import tpu as pltpu

N_DEV = 32
M = 8192
N_OUT = 4096
SEG = M // N_DEV


def kernel(x, w_mat):
    m, k_per = x.shape
    _, n = w_mat.shape

    def body(x_ref, w_ref, out_ref,
             acc_ref, rs_recv_ref, ag_send_ref, ag_recv_ref,
             rs_send_sem, rs_recv_sem, ag_send_sem, ag_recv_sem,
             rs_credit, ag_credit, copy_sem):
        my = lax.axis_index("i")
        left = lax.rem(my + N_DEV - 1, N_DEV)
        right = lax.rem(my + 1, N_DEV)

        barrier = pltpu.get_barrier_semaphore()
        pl.semaphore_signal(barrier, inc=1, device_id=(left,),
                            device_id_type=pl.DeviceIdType.MESH)
        pl.semaphore_signal(barrier, inc=1, device_id=(right,),
                            device_id_type=pl.DeviceIdType.MESH)
        pl.semaphore_wait(barrier, 2)

        pl.semaphore_signal(rs_credit, inc=1, device_id=(left,),
                            device_id_type=pl.DeviceIdType.MESH)
        pl.semaphore_signal(ag_credit, inc=1, device_id=(left,),
                            device_id_type=pl.DeviceIdType.MESH)

        def partial(seg):
            return jnp.dot(x_ref[pl.ds(seg * SEG, SEG), :], w_ref[...],
                           preferred_element_type=jnp.float32)

        acc_ref[...] = partial(my)

        def rs_step(t, carry):
            pl.semaphore_wait(rs_credit, 1)
            rdma = pltpu.make_async_remote_copy(
                src_ref=acc_ref, dst_ref=rs_recv_ref,
                send_sem=rs_send_sem, recv_sem=rs_recv_sem,
                device_id=(right,), device_id_type=pl.DeviceIdType.MESH,
            )
            rdma.start()
            seg = lax.rem(my - t - 1 + 2 * N_DEV, N_DEV)
            part = partial(seg)
            rdma.wait()
            acc_ref[...] = rs_recv_ref[...] + part

            @pl.when(t < N_DEV - 2)
            def _():
                pl.semaphore_signal(rs_credit, inc=1, device_id=(left,),
                                    device_id_type=pl.DeviceIdType.MESH)
            return carry

        lax.fori_loop(0, N_DEV - 1, rs_step, 0)

        my_seg = lax.rem(my + 1, N_DEV)
        ag_send_ref[...] = jnp.maximum(acc_ref[...], 0.0)
        copy = pltpu.make_async_copy(
            ag_send_ref, out_ref.at[pl.ds(my_seg * SEG, SEG), :], copy_sem)
        copy.start()
        copy.wait()

        def ag_step(t, carry):
            pl.semaphore_wait(ag_credit, 1)
            rdma = pltpu.make_async_remote_copy(
                src_ref=ag_send_ref, dst_ref=ag_recv_ref,
                send_sem=ag_send_sem, recv_sem=ag_recv_sem,
                device_id=(right,), device_id_type=pl.DeviceIdType.MESH,
            )
            rdma.start()
            rdma.wait()
            s_recv = lax.rem(my - t + 2 * N_DEV, N_DEV)
            ag_send_ref[...] = ag_recv_ref[...]
            cp = pltpu.make_async_copy(
                ag_recv_ref, out_ref.at[pl.ds(s_recv * SEG, SEG), :], copy_sem)
            cp.start()
            cp.wait()

            @pl.when(t < N_DEV - 2)
            def _():
                pl.semaphore_signal(ag_credit, inc=1, device_id=(left,),
                                    device_id_type=pl.DeviceIdType.MESH)
            return carry

        lax.fori_loop(0, N_DEV - 1, ag_step, 0)

    return pl.pallas_call(
        body,
        out_shape=jax.ShapeDtypeStruct((M, N_OUT), jnp.float32),
        in_specs=[
            pl.BlockSpec(memory_space=pltpu.VMEM),
            pl.BlockSpec(memory_space=pltpu.VMEM),
        ],
        out_specs=pl.BlockSpec(memory_space=pl.ANY),
        scratch_shapes=[
            pltpu.VMEM((SEG, N_OUT), jnp.float32),
            pltpu.VMEM((SEG, N_OUT), jnp.float32),
            pltpu.VMEM((SEG, N_OUT), jnp.float32),
            pltpu.VMEM((SEG, N_OUT), jnp.float32),
            pltpu.SemaphoreType.DMA,
            pltpu.SemaphoreType.DMA,
            pltpu.SemaphoreType.DMA,
            pltpu.SemaphoreType.DMA,
            pltpu.SemaphoreType.REGULAR,
            pltpu.SemaphoreType.REGULAR,
            pltpu.SemaphoreType.DMA,
        ],
        compiler_params=pltpu.CompilerParams(collective_id=0),
    )(x, w_mat)


# device time: 1959233 ns/iter; 1.7352x vs baseline; 1.7352x over previous
import jax
import jax.numpy as jnp
from jax import lax
from jax.experimental import pallas as pl
from jax.experimental.pallas import tpu as pltpu

N_DEV = 32
M = 8192
N_OUT = 4096
SEG = M // N_DEV


def kernel(x, w_mat):
    m, k_per = x.shape
    _, n = w_mat.shape

    def body(x_ref, w_ref, out_ref,
             acc_ref, stage_ref, rs_send_ref, rs_recv_ref,
             ag_send_ref, ag_recv_ref,
             rs_send_sem, rs_recv_sem, ag_send_sem, ag_recv_sem,
             rs_credit, ag_credit, copy_sem):
        my = lax.axis_index("i")
        left = lax.rem(my + N_DEV - 1, N_DEV)
        right = lax.rem(my + 1, N_DEV)

        barrier = pltpu.get_barrier_semaphore()
        pl.semaphore_signal(barrier, inc=1, device_id=(left,),
                            device_id_type=pl.DeviceIdType.MESH)
        pl.semaphore_signal(barrier, inc=1, device_id=(right,),
                            device_id_type=pl.DeviceIdType.MESH)
        pl.semaphore_wait(barrier, 2)

        pl.semaphore_signal(rs_credit, inc=1, device_id=(left,),
                            device_id_type=pl.DeviceIdType.MESH)
        pl.semaphore_signal(ag_credit, inc=1, device_id=(left,),
                            device_id_type=pl.DeviceIdType.MESH)

        def partial(seg):
            return jnp.dot(x_ref[pl.ds(seg * SEG, SEG), :], w_ref[...],
                           preferred_element_type=jnp.float32)

        def out_copy(src_ref, seg):
            return pltpu.make_async_copy(
                src_ref, out_ref.at[pl.ds(seg * SEG, SEG), :], copy_sem)

        acc_ref[...] = partial(my)
        rs_send_ref[...] = acc_ref[...].astype(jnp.bfloat16)

        def rs_step(t, carry):
            pl.semaphore_wait(rs_credit, 1)
            rdma = pltpu.make_async_remote_copy(
                src_ref=rs_send_ref, dst_ref=rs_recv_ref,
                send_sem=rs_send_sem, recv_sem=rs_recv_sem,
                device_id=(right,), device_id_type=pl.DeviceIdType.MESH,
            )
            rdma.start()
            seg = lax.rem(my - t - 1 + 2 * N_DEV, N_DEV)
            part = partial(seg)
            rdma.wait()
            new_acc = rs_recv_ref[...] + part
            acc_ref[...] = new_acc
            rs_send_ref[...] = new_acc.astype(jnp.bfloat16)

            @pl.when(t < N_DEV - 2)
            def _():
                pl.semaphore_signal(rs_credit, inc=1, device_id=(left,),
                                    device_id_type=pl.DeviceIdType.MESH)
            return carry

        lax.fori_loop(0, N_DEV - 1, rs_step, 0)

        my_seg = lax.rem(my + 1, N_DEV)
        final = jnp.maximum(acc_ref[...], 0.0)
        acc_ref[...] = final
        ag_send_ref[...] = final.astype(jnp.bfloat16)
        out_copy(acc_ref, my_seg).start()

        def ag_step(t, carry):
            pl.semaphore_wait(ag_credit, 1)
            rdma = pltpu.make_async_remote_copy(
                src_ref=ag_send_ref, dst_ref=ag_recv_ref,
                send_sem=ag_send_sem, recv_sem=ag_recv_sem,
                device_id=(right,), device_id_type=pl.DeviceIdType.MESH,
            )
            rdma.start()
            s_prev = lax.rem(my - t + 1 + 2 * N_DEV, N_DEV)
            out_copy(stage_ref, s_prev).wait()
            rdma.wait()
            v = ag_recv_ref[...]
            ag_send_ref[...] = v
            stage_ref[...] = v.astype(jnp.float32)

            @pl.when(t < N_DEV - 2)
            def _():
                pl.semaphore_signal(ag_credit, inc=1, device_id=(left,),
                                    device_id_type=pl.DeviceIdType.MESH)

            s_recv = lax.rem(my - t + 2 * N_DEV, N_DEV)
            out_copy(stage_ref, s_recv).start()
            return carry

        lax.fori_loop(0, N_DEV - 1, ag_step, 0)
        last_seg = lax.rem(my - N_DEV + 2 + 2 * N_DEV, N_DEV)
        out_copy(stage_ref, last_seg).wait()

    return pl.pallas_call(
        body,
        out_shape=jax.ShapeDtypeStruct((M, N_OUT), jnp.float32),
        in_specs=[
            pl.BlockSpec(memory_space=pltpu.VMEM),
            pl.BlockSpec(memory_space=pltpu.VMEM),
        ],
        out_specs=pl.BlockSpec(memory_space=pl.ANY),
        scratch_shapes=[
            pltpu.VMEM((SEG, N_OUT), jnp.float32),
            pltpu.VMEM((SEG, N_OUT), jnp.float32),
            pltpu.VMEM((SEG, N_OUT), jnp.bfloat16),
            pltpu.VMEM((SEG, N_OUT), jnp.bfloat16),
            pltpu.VMEM((SEG, N_OUT), jnp.bfloat16),
            pltpu.VMEM((SEG, N_OUT), jnp.bfloat16),
            pltpu.SemaphoreType.DMA,
            pltpu.SemaphoreType.DMA,
            pltpu.SemaphoreType.DMA,
            pltpu.SemaphoreType.DMA,
            pltpu.SemaphoreType.REGULAR,
            pltpu.SemaphoreType.REGULAR,
            pltpu.SemaphoreType.DMA,
        ],
        compiler_params=pltpu.CompilerParams(collective_id=0),
    )(x, w_mat)


# device time: 1661351 ns/iter; 2.0463x vs baseline; 1.1793x over previous
import jax
import jax.numpy as jnp
from jax import lax
from jax.experimental import pallas as pl
from jax.experimental.pallas import tpu as pltpu

N_DEV = 32
M = 8192
N_OUT = 4096
SEG = M // N_DEV
H = N_OUT // 2


def kernel(x, w_mat):
    def body(x_ref, w_ref, out_ref,
             acc_p, acc_m, stage_p, stage_m,
             rs_send_p, rs_recv_p, rs_send_m, rs_recv_m,
             ag_send_p, ag_recv_p, ag_send_m, ag_recv_m,
             rs_ssem_p, rs_rsem_p, rs_ssem_m, rs_rsem_m,
             ag_ssem_p, ag_rsem_p, ag_ssem_m, ag_rsem_m,
             rs_credit_p, rs_credit_m, ag_credit_p, ag_credit_m,
             copy_sem_p, copy_sem_m):
        my = lax.axis_index("i")
        left = lax.rem(my + N_DEV - 1, N_DEV)
        right = lax.rem(my + 1, N_DEV)
        MESH = pl.DeviceIdType.MESH

        barrier = pltpu.get_barrier_semaphore()
        pl.semaphore_signal(barrier, inc=1, device_id=(left,), device_id_type=MESH)
        pl.semaphore_signal(barrier, inc=1, device_id=(right,), device_id_type=MESH)
        pl.semaphore_wait(barrier, 2)

        pl.semaphore_signal(rs_credit_p, inc=1, device_id=(left,), device_id_type=MESH)
        pl.semaphore_signal(ag_credit_p, inc=1, device_id=(left,), device_id_type=MESH)
        pl.semaphore_signal(rs_credit_m, inc=1, device_id=(right,), device_id_type=MESH)
        pl.semaphore_signal(ag_credit_m, inc=1, device_id=(right,), device_id_type=MESH)

        def part_p(seg):
            return jnp.dot(x_ref[pl.ds(seg * SEG, SEG), :],
                           w_ref[:, pl.ds(0, H)],
                           preferred_element_type=jnp.float32)

        def part_m(seg):
            return jnp.dot(x_ref[pl.ds(seg * SEG, SEG), :],
                           w_ref[:, pl.ds(H, H)],
                           preferred_element_type=jnp.float32)

        def out_copy(src_ref, seg, col0, sem):
            return pltpu.make_async_copy(
                src_ref,
                out_ref.at[pl.ds(seg * SEG, SEG), pl.ds(col0, H)],
                sem)

        acc_p[...] = part_p(my)
        acc_m[...] = part_m(my)
        rs_send_p[...] = acc_p[...].astype(jnp.bfloat16)
        rs_send_m[...] = acc_m[...].astype(jnp.bfloat16)

        def rs_step(t, carry):
            pl.semaphore_wait(rs_credit_p, 1)
            pl.semaphore_wait(rs_credit_m, 1)
            rdma_p = pltpu.make_async_remote_copy(
                src_ref=rs_send_p, dst_ref=rs_recv_p,
                send_sem=rs_ssem_p, recv_sem=rs_rsem_p,
                device_id=(right,), device_id_type=MESH)
            rdma_m = pltpu.make_async_remote_copy(
                src_ref=rs_send_m, dst_ref=rs_recv_m,
                send_sem=rs_ssem_m, recv_sem=rs_rsem_m,
                device_id=(left,), device_id_type=MESH)
            rdma_p.start()
            rdma_m.start()
            seg_p = lax.rem(my - t - 1 + 2 * N_DEV, N_DEV)
            seg_m = lax.rem(my + t + 1, N_DEV)
            pp = part_p(seg_p)
            pm = part_m(seg_m)
            rdma_p.wait()
            new_p = rs_recv_p[...] + pp
            acc_p[...] = new_p
            rs_send_p[...] = new_p.astype(jnp.bfloat16)
            rdma_m.wait()
            new_m = rs_recv_m[...] + pm
            acc_m[...] = new_m
            rs_send_m[...] = new_m.astype(jnp.bfloat16)

            @pl.when(t < N_DEV - 2)
            def _():
                pl.semaphore_signal(rs_credit_p, inc=1, device_id=(left,),
                                    device_id_type=MESH)
                pl.semaphore_signal(rs_credit_m, inc=1, device_id=(right,),
                                    device_id_type=MESH)
            return carry

        lax.fori_loop(0, N_DEV - 1, rs_step, 0)

        fin_p = jnp.maximum(acc_p[...], 0.0)
        fin_m = jnp.maximum(acc_m[...], 0.0)
        acc_p[...] = fin_p
        acc_m[...] = fin_m
        ag_send_p[...] = fin_p.astype(jnp.bfloat16)
        ag_send_m[...] = fin_m.astype(jnp.bfloat16)
        out_copy(acc_p, right, 0, copy_sem_p).start()
        out_copy(acc_m, left, H, copy_sem_m).start()

        def ag_step(t, carry):
            pl.semaphore_wait(ag_credit_p, 1)
            pl.semaphore_wait(ag_credit_m, 1)
            rdma_p = pltpu.make_async_remote_copy(
                src_ref=ag_send_p, dst_ref=ag_recv_p,
                send_sem=ag_ssem_p, recv_sem=ag_rsem_p,
                device_id=(right,), device_id_type=MESH)
            rdma_m = pltpu.make_async_remote_copy(
                src_ref=ag_send_m, dst_ref=ag_recv_m,
                send_sem=ag_ssem_m, recv_sem=ag_rsem_m,
                device_id=(left,), device_id_type=MESH)
            rdma_p.start()
            rdma_m.start()
            s_prev_p = lax.rem(my - t + 1 + 2 * N_DEV, N_DEV)
            s_prev_m = lax.rem(my + t - 1 + 2 * N_DEV, N_DEV)
            out_copy(stage_p, s_prev_p, 0, copy_sem_p).wait()
            out_copy(stage_m, s_prev_m, H, copy_sem_m).wait()
            rdma_p.wait()
            v_p = ag_recv_p[...]
            ag_send_p[...] = v_p
            stage_p[...] = v_p.astype(jnp.float32)
            rdma_m.wait()
            v_m = ag_recv_m[...]
            ag_send_m[...] = v_m
            stage_m[...] = v_m.astype(jnp.float32)

            @pl.when(t < N_DEV - 2)
            def _():
                pl.semaphore_signal(ag_credit_p, inc=1, device_id=(left,),
                                    device_id_type=MESH)
                pl.semaphore_signal(ag_credit_m, inc=1, device_id=(right,),
                                    device_id_type=MESH)

            s_p = lax.rem(my - t + 2 * N_DEV, N_DEV)
            s_m = lax.rem(my + t, N_DEV)
            out_copy(stage_p, s_p, 0, copy_sem_p).start()
            out_copy(stage_m, s_m, H, copy_sem_m).start()
            return carry

        lax.fori_loop(0, N_DEV - 1, ag_step, 0)
        last_p = lax.rem(my - N_DEV + 2 + 2 * N_DEV, N_DEV)
        last_m = lax.rem(my + N_DEV - 2, N_DEV)
        out_copy(stage_p, last_p, 0, copy_sem_p).wait()
        out_copy(stage_m, last_m, H, copy_sem_m).wait()

    bf16 = jnp.bfloat16
    f32 = jnp.float32
    return pl.pallas_call(
        body,
        out_shape=jax.ShapeDtypeStruct((M, N_OUT), f32),
        in_specs=[
            pl.BlockSpec(memory_space=pltpu.VMEM),
            pl.BlockSpec(memory_space=pltpu.VMEM),
        ],
        out_specs=pl.BlockSpec(memory_space=pl.ANY),
        scratch_shapes=[
            pltpu.VMEM((SEG, H), f32),
            pltpu.VMEM((SEG, H), f32),
            pltpu.VMEM((SEG, H), f32),
            pltpu.VMEM((SEG, H), f32),
            pltpu.VMEM((SEG, H), bf16),
            pltpu.VMEM((SEG, H), bf16),
            pltpu.VMEM((SEG, H), bf16),
            pltpu.VMEM((SEG, H), bf16),
            pltpu.VMEM((SEG, H), bf16),
            pltpu.VMEM((SEG, H), bf16),
            pltpu.VMEM((SEG, H), bf16),
            pltpu.VMEM((SEG, H), bf16),
            pltpu.SemaphoreType.DMA,
            pltpu.SemaphoreType.DMA,
            pltpu.SemaphoreType.DMA,
            pltpu.SemaphoreType.DMA,
            pltpu.SemaphoreType.DMA,
            pltpu.SemaphoreType.DMA,
            pltpu.SemaphoreType.DMA,
            pltpu.SemaphoreType.DMA,
            pltpu.SemaphoreType.REGULAR,
            pltpu.SemaphoreType.REGULAR,
            pltpu.SemaphoreType.REGULAR,
            pltpu.SemaphoreType.REGULAR,
            pltpu.SemaphoreType.DMA,
            pltpu.SemaphoreType.DMA,
        ],
        compiler_params=pltpu.CompilerParams(collective_id=0),
    )(x, w_mat)


# device time: 978910 ns/iter; 3.4728x vs baseline; 1.6971x over previous
import jax
import jax.numpy as jnp
from jax import lax
from jax.experimental import pallas as pl
from jax.experimental.pallas import tpu as pltpu

N_DEV = 32
M = 8192
N_OUT = 4096
SEG = M // N_DEV
H = N_OUT // 2


def kernel(x, w_mat):
    def body(x_ref, w_ref, out_ref,
             acc_p, acc_m, stage_p, stage_m,
             rs_send_p, rs_recv_p, rs_send_m, rs_recv_m,
             ag_send_p, ag_recv_p, ag_send_m, ag_recv_m,
             rs_ssem_p, rs_rsem_p, rs_ssem_m, rs_rsem_m,
             ag_ssem_p, ag_rsem_p, ag_ssem_m, ag_rsem_m,
             rs_credit_p, rs_credit_m, ag_credit_p, ag_credit_m,
             copy_sem_p, copy_sem_m):
        my = lax.axis_index("i")
        MESH = pl.DeviceIdType.MESH

        def l2c(k):
            z = k // 8
            j = lax.rem(k, 8)
            y = j // 2
            xi = lax.rem(j, 2)
            x = jnp.where(lax.rem(y, 2) == 0, xi, 1 - xi)
            return x, y, z

        def c2l(x, y, z):
            return z * 8 + y * 2 + jnp.where(lax.rem(y, 2) == 0, x, 1 - x)

        def c2r(x, y, z):
            i = 3 - y
            r0 = 4 * y + jnp.where(lax.rem(y, 2) == 0, z, 3 - z)
            r1 = 16 + 4 * i + jnp.where(lax.rem(i, 2) == 0, z, 3 - z)
            return jnp.where(x == 0, r0, r1)

        def r2c(rr):
            x = jnp.where(rr < 16, 0, 1)
            y0 = lax.rem(rr, 16) // 4
            q = lax.rem(rr, 4)
            y = jnp.where(x == 0, y0, 3 - y0)
            z = jnp.where(lax.rem(y0, 2) == 0, q, 3 - q)
            return x, y, z

        r = c2r(*l2c(my))
        right = c2l(*r2c(lax.rem(r + 1, N_DEV)))
        left = c2l(*r2c(lax.rem(r + N_DEV - 1, N_DEV)))

        barrier = pltpu.get_barrier_semaphore()
        pl.semaphore_signal(barrier, inc=1, device_id=(left,), device_id_type=MESH)
        pl.semaphore_signal(barrier, inc=1, device_id=(right,), device_id_type=MESH)
        pl.semaphore_wait(barrier, 2)

        pl.semaphore_signal(rs_credit_p, inc=1, device_id=(left,), device_id_type=MESH)
        pl.semaphore_signal(ag_credit_p, inc=1, device_id=(left,), device_id_type=MESH)
        pl.semaphore_signal(rs_credit_m, inc=1, device_id=(right,), device_id_type=MESH)
        pl.semaphore_signal(ag_credit_m, inc=1, device_id=(right,), device_id_type=MESH)

        def part_p(seg):
            return jnp.dot(x_ref[pl.ds(seg * SEG, SEG), :],
                           w_ref[:, pl.ds(0, H)],
                           preferred_element_type=jnp.float32)

        def part_m(seg):
            return jnp.dot(x_ref[pl.ds(seg * SEG, SEG), :],
                           w_ref[:, pl.ds(H, H)],
                           preferred_element_type=jnp.float32)

        def out_copy(src_ref, seg, col0, sem):
            return pltpu.make_async_copy(
                src_ref,
                out_ref.at[pl.ds(seg * SEG, SEG), pl.ds(col0, H)],
                sem)

        acc_p[...] = part_p(r)
        acc_m[...] = part_m(r)
        rs_send_p[...] = acc_p[...].astype(jnp.bfloat16)
        rs_send_m[...] = acc_m[...].astype(jnp.bfloat16)

        def rs_step(t, carry):
            pl.semaphore_wait(rs_credit_p, 1)
            pl.semaphore_wait(rs_credit_m, 1)
            rdma_p = pltpu.make_async_remote_copy(
                src_ref=rs_send_p, dst_ref=rs_recv_p,
                send_sem=rs_ssem_p, recv_sem=rs_rsem_p,
                device_id=(right,), device_id_type=MESH)
            rdma_m = pltpu.make_async_remote_copy(
                src_ref=rs_send_m, dst_ref=rs_recv_m,
                send_sem=rs_ssem_m, recv_sem=rs_rsem_m,
                device_id=(left,), device_id_type=MESH)
            rdma_p.start()
            rdma_m.start()
            seg_p = lax.rem(r - t - 1 + 2 * N_DEV, N_DEV)
            seg_m = lax.rem(r + t + 1, N_DEV)
            pp = part_p(seg_p)
            pm = part_m(seg_m)
            rdma_p.wait()
            new_p = rs_recv_p[...] + pp
            acc_p[...] = new_p
            rs_send_p[...] = new_p.astype(jnp.bfloat16)
            rdma_m.wait()
            new_m = rs_recv_m[...] + pm
            acc_m[...] = new_m
            rs_send_m[...] = new_m.astype(jnp.bfloat16)

            @pl.when(t < N_DEV - 2)
            def _():
                pl.semaphore_signal(rs_credit_p, inc=1, device_id=(left,),
                                    device_id_type=MESH)
                pl.semaphore_signal(rs_credit_m, inc=1, device_id=(right,),
                                    device_id_type=MESH)
            return carry

        lax.fori_loop(0, N_DEV - 1, rs_step, 0)

        fin_p = jnp.maximum(acc_p[...], 0.0)
        fin_m = jnp.maximum(acc_m[...], 0.0)
        acc_p[...] = fin_p
        acc_m[...] = fin_m
        ag_send_p[...] = fin_p.astype(jnp.bfloat16)
        ag_send_m[...] = fin_m.astype(jnp.bfloat16)
        out_copy(acc_p, lax.rem(r + 1, N_DEV), 0, copy_sem_p).start()
        out_copy(acc_m, lax.rem(r + N_DEV - 1, N_DEV), H, copy_sem_m).start()

        def ag_step(t, carry):
            pl.semaphore_wait(ag_credit_p, 1)
            pl.semaphore_wait(ag_credit_m, 1)
            rdma_p = pltpu.make_async_remote_copy(
                src_ref=ag_send_p, dst_ref=ag_recv_p,
                send_sem=ag_ssem_p, recv_sem=ag_rsem_p,
                device_id=(right,), device_id_type=MESH)
            rdma_m = pltpu.make_async_remote_copy(
                src_ref=ag_send_m, dst_ref=ag_recv_m,
                send_sem=ag_ssem_m, recv_sem=ag_rsem_m,
                device_id=(left,), device_id_type=MESH)
            rdma_p.start()
            rdma_m.start()
            s_prev_p = lax.rem(r - t + 1 + 2 * N_DEV, N_DEV)
            s_prev_m = lax.rem(r + t - 1 + 2 * N_DEV, N_DEV)
            out_copy(stage_p, s_prev_p, 0, copy_sem_p).wait()
            out_copy(stage_m, s_prev_m, H, copy_sem_m).wait()
            rdma_p.wait()
            v_p = ag_recv_p[...]
            ag_send_p[...] = v_p
            stage_p[...] = v_p.astype(jnp.float32)
            rdma_m.wait()
            v_m = ag_recv_m[...]
            ag_send_m[...] = v_m
            stage_m[...] = v_m.astype(jnp.float32)

            @pl.when(t < N_DEV - 2)
            def _():
                pl.semaphore_signal(ag_credit_p, inc=1, device_id=(left,),
                                    device_id_type=MESH)
                pl.semaphore_signal(ag_credit_m, inc=1, device_id=(right,),
                                    device_id_type=MESH)

            s_p = lax.rem(r - t + 2 * N_DEV, N_DEV)
            s_m = lax.rem(r + t, N_DEV)
            out_copy(stage_p, s_p, 0, copy_sem_p).start()
            out_copy(stage_m, s_m, H, copy_sem_m).start()
            return carry

        lax.fori_loop(0, N_DEV - 1, ag_step, 0)
        last_p = lax.rem(r - N_DEV + 2 + 2 * N_DEV, N_DEV)
        last_m = lax.rem(r + N_DEV - 2, N_DEV)
        out_copy(stage_p, last_p, 0, copy_sem_p).wait()
        out_copy(stage_m, last_m, H, copy_sem_m).wait()

    bf16 = jnp.bfloat16
    f32 = jnp.float32
    return pl.pallas_call(
        body,
        out_shape=jax.ShapeDtypeStruct((M, N_OUT), f32),
        in_specs=[
            pl.BlockSpec(memory_space=pltpu.VMEM),
            pl.BlockSpec(memory_space=pltpu.VMEM),
        ],
        out_specs=pl.BlockSpec(memory_space=pl.ANY),
        scratch_shapes=[
            pltpu.VMEM((SEG, H), f32),
            pltpu.VMEM((SEG, H), f32),
            pltpu.VMEM((SEG, H), f32),
            pltpu.VMEM((SEG, H), f32),
            pltpu.VMEM((SEG, H), bf16),
            pltpu.VMEM((SEG, H), bf16),
            pltpu.VMEM((SEG, H), bf16),
            pltpu.VMEM((SEG, H), bf16),
            pltpu.VMEM((SEG, H), bf16),
            pltpu.VMEM((SEG, H), bf16),
            pltpu.VMEM((SEG, H), bf16),
            pltpu.VMEM((SEG, H), bf16),
            pltpu.SemaphoreType.DMA,
            pltpu.SemaphoreType.DMA,
            pltpu.SemaphoreType.DMA,
            pltpu.SemaphoreType.DMA,
            pltpu.SemaphoreType.DMA,
            pltpu.SemaphoreType.DMA,
            pltpu.SemaphoreType.DMA,
            pltpu.SemaphoreType.DMA,
            pltpu.SemaphoreType.REGULAR,
            pltpu.SemaphoreType.REGULAR,
            pltpu.SemaphoreType.REGULAR,
            pltpu.SemaphoreType.REGULAR,
            pltpu.SemaphoreType.DMA,
            pltpu.SemaphoreType.DMA,
        ],
        compiler_params=pltpu.CompilerParams(collective_id=0),
    )(x, w_mat)


# device time: 798614 ns/iter; 4.2568x vs baseline; 1.2258x over previous
import jax
import jax.numpy as jnp
from jax import lax
from jax.experimental import pallas as pl
from jax.experimental.pallas import tpu as pltpu

N_DEV = 32
M = 8192
N_OUT = 4096
SEG = M // N_DEV
HQ = N_OUT // 4


def kernel(x, w_mat):
    def body(x_ref, w_ref, out_ref,
             rs_send, rs_recv, ag_send, ag_recv, stage,
             rs_ssem, rs_rsem, ag_ssem, ag_rsem,
             rs_credit, ag_credit, copy_sem):
        my = lax.axis_index("i")
        MESH = pl.DeviceIdType.MESH

        def l2c(k):
            z = k // 8
            j = lax.rem(k, 8)
            y = j // 2
            xi = lax.rem(j, 2)
            xx = jnp.where(lax.rem(y, 2) == 0, xi, 1 - xi)
            return xx, y, z

        def c2l(xx, y, z):
            return z * 8 + y * 2 + jnp.where(lax.rem(y, 2) == 0, xx, 1 - xx)

        def c2r(xx, y, z):
            i = 3 - y
            r0 = 4 * y + jnp.where(lax.rem(y, 2) == 0, z, 3 - z)
            r1 = 16 + 4 * i + jnp.where(lax.rem(i, 2) == 0, z, 3 - z)
            return jnp.where(xx == 0, r0, r1)

        def r2c(rr):
            xx = jnp.where(rr < 16, 0, 1)
            y0 = lax.rem(rr, 16) // 4
            q = lax.rem(rr, 4)
            y = jnp.where(xx == 0, y0, 3 - y0)
            z = jnp.where(lax.rem(y0, 2) == 0, q, 3 - q)
            return xx, y, z

        r = c2r(*l2c(my))
        rnext = c2l(*r2c(lax.rem(r + 1, N_DEV)))
        rprev = c2l(*r2c(lax.rem(r + N_DEV - 1, N_DEV)))

        def mod32(v):
            return lax.rem(v + 4 * N_DEV, N_DEV)

        RINGS = [
            dict(ri=0, col0=0 * HQ, sgn=-1),
            dict(ri=2, col0=2 * HQ, sgn=+1),
            dict(ri=1, col0=1 * HQ, sgn=-1),
            dict(ri=3, col0=3 * HQ, sgn=+1),
        ]
        for cfg in RINGS:
            cfg["tgt"] = rnext if cfg["sgn"] < 0 else rprev
            cfg["ups"] = rprev if cfg["sgn"] < 0 else rnext

        barrier = pltpu.get_barrier_semaphore()
        pl.semaphore_signal(barrier, inc=1, device_id=(rprev,), device_id_type=MESH)
        pl.semaphore_signal(barrier, inc=1, device_id=(rnext,), device_id_type=MESH)
        pl.semaphore_wait(barrier, 2)

        for cfg in RINGS:
            pl.semaphore_signal(rs_credit.at[cfg["ri"]], inc=1,
                                device_id=(cfg["ups"],), device_id_type=MESH)
            pl.semaphore_signal(ag_credit.at[cfg["ri"]], inc=1,
                                device_id=(cfg["ups"],), device_id_type=MESH)

        def part(seg, col0):
            return jnp.dot(x_ref[pl.ds(seg * SEG, SEG), :],
                           w_ref[:, pl.ds(col0, HQ)],
                           preferred_element_type=jnp.float32)

        def rs_rdma(cfg):
            return pltpu.make_async_remote_copy(
                src_ref=rs_send.at[cfg["ri"]], dst_ref=rs_recv.at[cfg["ri"]],
                send_sem=rs_ssem.at[cfg["ri"]], recv_sem=rs_rsem.at[cfg["ri"]],
                device_id=(cfg["tgt"],), device_id_type=MESH)

        def ag_rdma(cfg):
            return pltpu.make_async_remote_copy(
                src_ref=ag_send.at[cfg["ri"]], dst_ref=ag_recv.at[cfg["ri"]],
                send_sem=ag_ssem.at[cfg["ri"]], recv_sem=ag_rsem.at[cfg["ri"]],
                device_id=(cfg["tgt"],), device_id_type=MESH)

        def out_copy(src_ref, seg, col0, sem):
            return pltpu.make_async_copy(
                src_ref,
                out_ref.at[pl.ds(seg * SEG, SEG), pl.ds(col0, HQ)],
                sem)

        for cfg in RINGS:
            rs_send[cfg["ri"]] = part(r, cfg["col0"]).astype(jnp.bfloat16)
            pl.semaphore_wait(rs_credit.at[cfg["ri"]], 1)
            rs_rdma(cfg).start()

        def rs_step(t, carry):
            parts = [part(mod32(r + cfg["sgn"] * (t + 1)), cfg["col0"])
                     for cfg in RINGS]
            for cfg, p in zip(RINGS, parts):
                ri = cfg["ri"]
                rdma = rs_rdma(cfg)
                rdma.wait_recv()
                new = rs_recv[ri][...] + p
                rdma.wait_send()
                rs_send[ri] = new.astype(jnp.bfloat16)
                pl.semaphore_signal(rs_credit.at[ri], inc=1,
                                    device_id=(cfg["ups"],), device_id_type=MESH)
                pl.semaphore_wait(rs_credit.at[ri], 1)
                rdma.start()
            return carry

        lax.fori_loop(0, N_DEV - 2, rs_step, 0)

        for cfg in RINGS:
            ri = cfg["ri"]
            rdma = rs_rdma(cfg)
            rdma.wait_recv()
            p = part(mod32(r + cfg["sgn"] * (N_DEV - 1)), cfg["col0"])
            fin = jnp.maximum(rs_recv[ri][...] + p, 0.0)
            rdma.wait_send()
            stage[ri] = fin
            ag_send[ri] = fin.astype(jnp.bfloat16)
            own = mod32(r - cfg["sgn"])
            out_copy(stage.at[ri], own, cfg["col0"], copy_sem.at[ri]).start()
            pl.semaphore_wait(ag_credit.at[ri], 1)
            ag_rdma(cfg).start()

        def ag_step(t, carry):
            for cfg in RINGS:
                ri = cfg["ri"]
                rdma = ag_rdma(cfg)
                rdma.wait_recv()
                s_prev = mod32(r + cfg["sgn"] * (t - 1))
                out_copy(stage.at[ri], s_prev, cfg["col0"], copy_sem.at[ri]).wait()
                rdma.wait_send()
                v = ag_recv[ri][...]
                ag_send[ri] = v
                stage[ri] = v.astype(jnp.float32)
                pl.semaphore_signal(ag_credit.at[ri], inc=1,
                                    device_id=(cfg["ups"],), device_id_type=MESH)
                pl.semaphore_wait(ag_credit.at[ri], 1)
                rdma.start()
                s = mod32(r + cfg["sgn"] * t)
                out_copy(stage.at[ri], s, cfg["col0"], copy_sem.at[ri]).start()
            return carry

        lax.fori_loop(0, N_DEV - 2, ag_step, 0)

        for cfg in RINGS:
            ri = cfg["ri"]
            t = N_DEV - 2
            rdma = ag_rdma(cfg)
            rdma.wait_recv()
            out_copy(stage.at[ri], mod32(r + cfg["sgn"] * (t - 1)), cfg["col0"],
                     copy_sem.at[ri]).wait()
            rdma.wait_send()
            stage[ri] = ag_recv[ri][...].astype(jnp.float32)
            s = mod32(r + cfg["sgn"] * t)
            out_copy(stage.at[ri], s, cfg["col0"], copy_sem.at[ri]).start()
        for cfg in RINGS:
            ri = cfg["ri"]
            s = mod32(r + cfg["sgn"] * (N_DEV - 2))
            out_copy(stage.at[ri], s, cfg["col0"], copy_sem.at[ri]).wait()

    bf16 = jnp.bfloat16
    f32 = jnp.float32
    return pl.pallas_call(
        body,
        out_shape=jax.ShapeDtypeStruct((M, N_OUT), f32),
        in_specs=[
            pl.BlockSpec(memory_space=pltpu.VMEM),
            pl.BlockSpec(memory_space=pltpu.VMEM),
        ],
        out_specs=pl.BlockSpec(memory_space=pl.ANY),
        scratch_shapes=[
            pltpu.VMEM((4, SEG, HQ), bf16),
            pltpu.VMEM((4, SEG, HQ), bf16),
            pltpu.VMEM((4, SEG, HQ), bf16),
            pltpu.VMEM((4, SEG, HQ), bf16),
            pltpu.VMEM((4, SEG, HQ), f32),
            pltpu.SemaphoreType.DMA((4,)),
            pltpu.SemaphoreType.DMA((4,)),
            pltpu.SemaphoreType.DMA((4,)),
            pltpu.SemaphoreType.DMA((4,)),
            pltpu.SemaphoreType.REGULAR((4,)),
            pltpu.SemaphoreType.REGULAR((4,)),
            pltpu.SemaphoreType.DMA((4,)),
        ],
        compiler_params=pltpu.CompilerParams(collective_id=0),
    )(x, w_mat)
